# baseline (speedup 1.0000x reference)
import functools
import jax
import jax.numpy as jnp
from jax.experimental import pallas as pl

H = 4
C = 8
HID = 32
ROWS = 400

_INTERPRET = False


def _lrelu(v):
    return jnp.where(v >= 0, v, 0.2 * v)


def _elu(v):
    return jnp.where(v > 0, v, jnp.exp(jnp.minimum(v, 0.0)) - 1.0)


def _dense1_body(x_ref, wl_ref, bl_ref, wg_ref, asm_ref, adm_ref, wl1_ref,
                 bl1_ref, h1_o, xw_o, as_o, ad_o, lin_o):
    h1 = jnp.maximum(x_ref[...] @ wl_ref[...] + bl_ref[...], 0.0)
    xw = h1 @ wg_ref[...]
    h1_o[...] = h1
    xw_o[...] = xw
    as_o[...] = xw @ asm_ref[...]
    ad_o[...] = xw @ adm_ref[...]
    lin_o[...] = h1 @ wl1_ref[...] + bl1_ref[...]


def _merge(num_ref, den_ref, as_ref, ad_ref, xw_ref, lin_ref, bg_ref, bm_ref):
    a = as_ref[...] + ad_ref[...]
    e_self = jnp.exp(_lrelu(a))
    den = (den_ref[...] + e_self) @ bm_ref[...]
    num = num_ref[...] + (e_self @ bm_ref[...]) * xw_ref[...]
    return _elu(num / den + bg_ref[...] + lin_ref[...])


def _dense2_body(num_ref, den_ref, as_ref, ad_ref, xw_ref, lin_ref, bg_ref,
                 bm_ref, wg_ref, asm_ref, adm_ref, wl_ref, bl_ref,
                 h2_o, xw_o, as_o, ad_o, lin_o):
    h2 = _merge(num_ref, den_ref, as_ref, ad_ref, xw_ref, lin_ref, bg_ref,
                bm_ref)
    xw = h2 @ wg_ref[...]
    h2_o[...] = h2
    xw_o[...] = xw
    as_o[...] = xw @ asm_ref[...]
    ad_o[...] = xw @ adm_ref[...]
    lin_o[...] = h2 @ wl_ref[...] + bl_ref[...]


def _dense3_body(h1_ref, h2_ref, num_ref, den_ref, as_ref, ad_ref, xw_ref,
                 lin_ref, bg_ref, bm_ref, wf1_ref, wf2_ref, wf3_ref, bf_ref,
                 wo_ref, bo_ref, out_o):
    h3 = _merge(num_ref, den_ref, as_ref, ad_ref, xw_ref, lin_ref, bg_ref,
                bm_ref)
    t = jnp.maximum(h1_ref[...] @ wf1_ref[...] + h2_ref[...] @ wf2_ref[...]
                    + h3 @ wf3_ref[...] + bf_ref[...], 0.0)
    out_o[...] = jax.nn.sigmoid(t @ wo_ref[...] + bo_ref[...])


def _row_spec(cols):
    return pl.BlockSpec((ROWS, cols), lambda i: (i, 0))


def _full_spec(shape):
    nd = len(shape)
    return pl.BlockSpec(shape, lambda i: (0,) * nd)


def _att_mat(a):
    m = jnp.zeros((H * C, 16), a.dtype)
    return m.at[jnp.arange(H * C), jnp.repeat(jnp.arange(H), C)].set(
        a.reshape(-1))


def _edge_phase(as16, ad16, xw, dst_sorted_src, n):
    src, dst = dst_sorted_src
    e = jnp.exp(_lrelu(as16[src, :H] + ad16[dst, :H]))
    den = jax.ops.segment_sum(e, dst, num_segments=n)
    num = jax.ops.segment_sum(xw[src] * jnp.repeat(e, C, axis=1), dst,
                              num_segments=n)
    return num, jnp.pad(den, ((0, 0), (0, 16 - H)))


def kernel(x, edge_index, W_line, b_line, W_g1, a_src1, a_dst1, b_g1, W_l1,
           b_l1, W_g2, a_src2, a_dst2, b_g2, W_l2, b_l2, W_fc, b_fc, W_out,
           b_out):
    n = x.shape[0]
    d_in = x.shape[1]
    nblk = n // ROWS
    f32 = jnp.float32

    asm1, adm1 = _att_mat(a_src1), _att_mat(a_dst1)
    asm2, adm2 = _att_mat(a_src2), _att_mat(a_dst2)
    bmat = jnp.zeros((16, H * C), f32).at[
        jnp.repeat(jnp.arange(H), C), jnp.arange(H * C)].set(1.0)
    bl2d = b_line.reshape(1, -1)
    bl1_2d, bl2_2d = b_l1.reshape(1, -1), b_l2.reshape(1, -1)
    bg1_2d, bg2_2d = b_g1.reshape(1, -1), b_g2.reshape(1, -1)
    bf2d, bo2d = b_fc.reshape(1, -1), b_out.reshape(1, -1)
    wf1, wf2, wf3 = W_fc[:HID], W_fc[HID:2 * HID], W_fc[2 * HID:]

    node_out = lambda cols: (jax.ShapeDtypeStruct((n, cols), f32),
                             _row_spec(cols))

    shapes1 = [node_out(HID), node_out(HID), node_out(16), node_out(16),
               node_out(HID)]
    h1, xw1, as1, ad1, lin1 = pl.pallas_call(
        _dense1_body,
        grid=(nblk,),
        in_specs=[_row_spec(d_in), _full_spec(W_line.shape),
                  _full_spec((1, HID)), _full_spec(W_g1.shape),
                  _full_spec((HID, 16)), _full_spec((HID, 16)),
                  _full_spec(W_l1.shape), _full_spec((1, HID))],
        out_specs=[s for _, s in shapes1],
        out_shape=[s for s, _ in shapes1],
        interpret=_INTERPRET,
    )(x, W_line, bl2d, W_g1, asm1, adm1, W_l1, bl1_2d)

    src, dst = edge_index[0], edge_index[1]
    num1, den1 = _edge_phase(as1, ad1, xw1, (src, dst), n)

    h2, xw2, as2, ad2, lin2 = pl.pallas_call(
        _dense2_body,
        grid=(nblk,),
        in_specs=[_row_spec(HID), _row_spec(16), _row_spec(16), _row_spec(16),
                  _row_spec(HID), _row_spec(HID), _full_spec((1, HID)),
                  _full_spec((16, HID)), _full_spec(W_g2.shape),
                  _full_spec((HID, 16)), _full_spec((HID, 16)),
                  _full_spec(W_l2.shape), _full_spec((1, HID))],
        out_specs=[s for _, s in shapes1],
        out_shape=[s for s, _ in shapes1],
        interpret=_INTERPRET,
    )(num1, den1, as1, ad1, xw1, lin1, bg1_2d, bmat, W_g2, asm2, adm2, W_l2,
      bl2_2d)

    num2, den2 = _edge_phase(as2, ad2, xw2, (src, dst), n)

    out = pl.pallas_call(
        _dense3_body,
        grid=(nblk,),
        in_specs=[_row_spec(HID), _row_spec(HID), _row_spec(HID),
                  _row_spec(16), _row_spec(16), _row_spec(16), _row_spec(HID),
                  _row_spec(HID), _full_spec((1, HID)), _full_spec((16, HID)),
                  _full_spec((HID, 16)), _full_spec((HID, 16)),
                  _full_spec((HID, 16)), _full_spec((1, 16)),
                  _full_spec((16, 1)), _full_spec((1, 1))],
        out_specs=[_row_spec(1)],
        out_shape=[jax.ShapeDtypeStruct((n, 1), f32)],
        interpret=_INTERPRET,
    )(h1, h2, num2, den2, as2, ad2, xw2, lin2, bg2_2d, bmat, wf1, wf2, wf3,
      bf2d, W_out, bo2d)[0]

    return out.reshape(-1)
